# baseline (speedup 1.0000x reference)
import functools

import jax
import jax.numpy as jnp
from jax import lax
from jax.experimental import pallas as pl
from jax.experimental.pallas import tpu as pltpu
from jax.experimental.pallas import tpu_sc as plsc

BATCH, SEQ, DIM = 4, 8192, 768
NC, NS = 2, 16
NW = NC * NS
S_PER_W = SEQ // NW
CHUNK = 32
PCHUNKS = S_PER_W // CHUNK
NITER = PCHUNKS * BATCH
CHUNK_EL = CHUNK * DIM
UNROLL = 16

_mesh = plsc.VectorSubcoreMesh(core_axis_name="c", subcore_axis_name="s")


@functools.partial(
    pl.kernel,
    mesh=_mesh,
    out_type=jax.ShapeDtypeStruct((BATCH, SEQ, DIM), jnp.float32),
    scratch_types=[
        pltpu.VMEM((2, CHUNK_EL), jnp.float32),
        pltpu.VMEM((CHUNK_EL,), jnp.float32),
        pltpu.VMEM((2, CHUNK_EL), jnp.float32),
        pltpu.SemaphoreType.DMA,
        pltpu.SemaphoreType.DMA,
        pltpu.SemaphoreType.DMA,
        pltpu.SemaphoreType.DMA,
        pltpu.SemaphoreType.DMA,
    ],
)
def _pos_add(x_hbm, pos_hbm, out_hbm, xbuf, pbuf, obuf,
             xsem0, xsem1, psem, osem0, osem1):
    wid = lax.axis_index("s") * NC + lax.axis_index("c")
    s0 = wid * S_PER_W

    xsems = (xsem0, xsem1)
    osems = (osem0, osem1)

    def rows_loop(fn):
        def body(r, _):
            fn(r)
            return 0
        lax.fori_loop(0, CHUNK, body, 0, unroll=4)

    def split(i):
        return i // BATCH, lax.rem(i, BATCH)

    def issue_x(i, p):
        pc, b = split(i)
        r0 = s0 + pc * CHUNK
        rows_loop(lambda r: pltpu.async_copy(
            x_hbm.at[b, r0 + r, :],
            xbuf.at[p, pl.ds(r * DIM, DIM)], xsems[p]))

    def wait_x(i, p):
        pc, b = split(i)
        r0 = s0 + pc * CHUNK
        rows_loop(lambda r: pltpu.make_async_copy(
            x_hbm.at[b, r0 + r, :],
            xbuf.at[p, pl.ds(r * DIM, DIM)], xsems[p]).wait())

    def issue_p(pc):
        r0 = s0 + pc * CHUNK
        rows_loop(lambda r: pltpu.async_copy(
            pos_hbm.at[r0 + r, :], pbuf.at[pl.ds(r * DIM, DIM)], psem))

    def wait_p(pc):
        r0 = s0 + pc * CHUNK
        rows_loop(lambda r: pltpu.make_async_copy(
            pos_hbm.at[r0 + r, :], pbuf.at[pl.ds(r * DIM, DIM)], psem).wait())

    def issue_o(i, p):
        pc, b = split(i)
        r0 = s0 + pc * CHUNK
        rows_loop(lambda r: pltpu.async_copy(
            obuf.at[p, pl.ds(r * DIM, DIM)],
            out_hbm.at[b, r0 + r, :], osems[p]))

    def wait_o(i, p):
        pc, b = split(i)
        r0 = s0 + pc * CHUNK
        rows_loop(lambda r: pltpu.make_async_copy(
            obuf.at[p, pl.ds(r * DIM, DIM)],
            out_hbm.at[b, r0 + r, :], osems[p]).wait())

    issue_p(0)
    issue_x(0, 0)

    def chunk_pair(k, _):
        for p in (0, 1):
            i = 2 * k + p
            pc, b = split(i)
            if p == 0:
                issue_x(i + 1, 1)
            else:
                @pl.when(i + 1 < NITER)
                def _():
                    issue_x(i + 1, 0)
            wait_x(i, p)

            @pl.when(b == 0)
            def _():
                wait_p(pc)

            @pl.when(i >= 2)
            def _():
                wait_o(i - 2, p)

            xb = xbuf.at[p]
            ob = obuf.at[p]

            @plsc.parallel_loop(0, CHUNK_EL, 16, unroll=UNROLL)
            def _(o, xb=xb, ob=ob):
                ob[pl.ds(o, 16)] = xb[pl.ds(o, 16)] + pbuf[pl.ds(o, 16)]

            issue_o(i, p)

            @pl.when((b == BATCH - 1) & (pc + 1 < PCHUNKS))
            def _():
                issue_p(pc + 1)
        return 0

    lax.fori_loop(0, NITER // 2, chunk_pair, 0)

    wait_o(NITER - 2, 0)
    wait_o(NITER - 1, 1)


def kernel(x, pos_emb):
    return _pos_add(x, pos_emb)


# speedup vs baseline: 1.0520x; 1.0520x over previous
import functools

import jax
import jax.numpy as jnp
from jax import lax
from jax.experimental import pallas as pl
from jax.experimental.pallas import tpu as pltpu
from jax.experimental.pallas import tpu_sc as plsc

BATCH, SEQ, DIM = 4, 8192, 768
NC, NS = 2, 16
NW = NC * NS
S_PER_W = SEQ // NW
CHUNK = 32
PCHUNKS = S_PER_W // CHUNK
NITER = PCHUNKS * BATCH
CHUNK_EL = CHUNK * DIM
UNROLL = 8

_mesh = plsc.VectorSubcoreMesh(core_axis_name="c", subcore_axis_name="s")


@functools.partial(
    pl.kernel,
    mesh=_mesh,
    out_type=jax.ShapeDtypeStruct((BATCH, SEQ, DIM), jnp.float32),
    scratch_types=[
        pltpu.VMEM((2, CHUNK_EL), jnp.float32),
        pltpu.VMEM((CHUNK_EL,), jnp.float32),
        pltpu.VMEM((2, CHUNK_EL), jnp.float32),
        pltpu.SemaphoreType.DMA,
        pltpu.SemaphoreType.DMA,
        pltpu.SemaphoreType.DMA,
        pltpu.SemaphoreType.DMA,
        pltpu.SemaphoreType.DMA,
    ],
)
def _pos_add(x_hbm, pos_hbm, dummy_hbm, out_hbm, xbuf, pbuf, obuf,
             xsem0, xsem1, psem, osem0, osem1):
    wid = lax.axis_index("s") * NC + lax.axis_index("c")
    s0 = wid * S_PER_W

    xsems = (xsem0, xsem1)
    osems = (osem0, osem1)

    def rows_loop(fn):
        def body(r, _):
            fn(r)
            return 0
        lax.fori_loop(0, CHUNK, body, 0)

    def split(i):
        return i // BATCH, lax.rem(i, BATCH)

    def issue_x(i, p):
        pc, b = split(i)
        r0 = s0 + pc * CHUNK
        rows_loop(lambda r: pltpu.async_copy(
            x_hbm.at[b, r0 + r, :],
            xbuf.at[p, pl.ds(r * DIM, DIM)], xsems[p]))

    def wait_x(p):
        pltpu.make_async_copy(dummy_hbm, xbuf.at[p], xsems[p]).wait()

    def issue_p(pc):
        r0 = s0 + pc * CHUNK
        rows_loop(lambda r: pltpu.async_copy(
            pos_hbm.at[r0 + r, :], pbuf.at[pl.ds(r * DIM, DIM)], psem))

    def wait_p():
        pltpu.make_async_copy(dummy_hbm, pbuf, psem).wait()

    def issue_o(i, p):
        pc, b = split(i)
        r0 = s0 + pc * CHUNK
        rows_loop(lambda r: pltpu.async_copy(
            obuf.at[p, pl.ds(r * DIM, DIM)],
            out_hbm.at[b, r0 + r, :], osems[p]))

    def wait_o(p):
        pltpu.make_async_copy(obuf.at[p], dummy_hbm, osems[p]).wait()

    issue_p(0)
    issue_x(0, 0)

    def chunk_pair(k, _):
        for p in (0, 1):
            i = 2 * k + p
            pc, b = split(i)
            if p == 0:
                issue_x(i + 1, 1)
            else:
                @pl.when(i + 1 < NITER)
                def _():
                    issue_x(i + 1, 0)
            wait_x(p)

            @pl.when(b == 0)
            def _():
                wait_p()

            @pl.when(i >= 2)
            def _():
                wait_o(p)

            xb = xbuf.at[p]
            ob = obuf.at[p]

            @plsc.parallel_loop(0, CHUNK_EL, 16, unroll=UNROLL)
            def _(o, xb=xb, ob=ob):
                ob[pl.ds(o, 16)] = xb[pl.ds(o, 16)] + pbuf[pl.ds(o, 16)]

            issue_o(i, p)

            @pl.when((b == BATCH - 1) & (pc + 1 < PCHUNKS))
            def _():
                issue_p(pc + 1)
        return 0

    lax.fori_loop(0, NITER // 2, chunk_pair, 0)

    wait_o(0)
    wait_o(1)


def kernel(x, pos_emb):
    dummy = jnp.zeros((CHUNK_EL,), jnp.float32)
    return _pos_add(x, pos_emb, dummy)


# speedup vs baseline: 1.1977x; 1.1385x over previous
import functools

import jax
import jax.numpy as jnp
from jax import lax
from jax.experimental import pallas as pl
from jax.experimental.pallas import tpu as pltpu
from jax.experimental.pallas import tpu_sc as plsc

BATCH, SEQ, DIM = 4, 8192, 768
NC, NS = 2, 16
NW = NC * NS
S_PER_W = SEQ // NW
CHUNK = 16
PCHUNKS = S_PER_W // CHUNK
NITER = PCHUNKS * BATCH
CHUNK_EL = CHUNK * DIM
UNROLL = 8
XDEPTH = 4

_mesh = plsc.VectorSubcoreMesh(core_axis_name="c", subcore_axis_name="s")


@functools.partial(
    pl.kernel,
    mesh=_mesh,
    out_type=jax.ShapeDtypeStruct((BATCH, SEQ, DIM), jnp.float32),
    scratch_types=[
        pltpu.VMEM((XDEPTH, CHUNK_EL), jnp.float32),
        pltpu.VMEM((2, CHUNK_EL), jnp.float32),
        pltpu.VMEM((2, CHUNK_EL), jnp.float32),
        pltpu.SemaphoreType.DMA,
        pltpu.SemaphoreType.DMA,
        pltpu.SemaphoreType.DMA,
        pltpu.SemaphoreType.DMA,
        pltpu.SemaphoreType.DMA,
        pltpu.SemaphoreType.DMA,
        pltpu.SemaphoreType.DMA,
        pltpu.SemaphoreType.DMA,
    ],
)
def _pos_add(x_hbm, pos_hbm, dummy_hbm, out_hbm, xbuf, pbuf, obuf,
             xsem0, xsem1, xsem2, xsem3, psem0, psem1, osem0, osem1):
    wid = lax.axis_index("s") * NC + lax.axis_index("c")
    s0 = wid * S_PER_W

    xsems = (xsem0, xsem1, xsem2, xsem3)
    psems = (psem0, psem1)
    osems = (osem0, osem1)

    def rows_loop(fn):
        def body(r, _):
            fn(r)
            return 0
        lax.fori_loop(0, CHUNK, body, 0)

    def split(i):
        return i // BATCH, lax.rem(i, BATCH)

    def issue_x(i, xi):
        pc, b = split(i)
        r0 = s0 + pc * CHUNK
        rows_loop(lambda r: pltpu.async_copy(
            x_hbm.at[b, r0 + r, :],
            xbuf.at[xi, pl.ds(r * DIM, DIM)], xsems[xi]))

    def wait_x(xi):
        pltpu.make_async_copy(dummy_hbm, xbuf.at[xi], xsems[xi]).wait()

    def issue_p(pc, pp):
        r0 = s0 + pc * CHUNK
        rows_loop(lambda r: pltpu.async_copy(
            pos_hbm.at[r0 + r, :], pbuf.at[pp, pl.ds(r * DIM, DIM)], psems[pp]))

    def wait_p(pp):
        pltpu.make_async_copy(dummy_hbm, pbuf.at[pp], psems[pp]).wait()

    def issue_o(i, oi):
        pc, b = split(i)
        r0 = s0 + pc * CHUNK
        rows_loop(lambda r: pltpu.async_copy(
            obuf.at[oi, pl.ds(r * DIM, DIM)],
            out_hbm.at[b, r0 + r, :], osems[oi]))

    def wait_o(oi):
        pltpu.make_async_copy(obuf.at[oi], dummy_hbm, osems[oi]).wait()

    issue_p(0, 0)
    for j in range(XDEPTH - 1):
        issue_x(j, j)

    def chunk_group(k, _):
        for j in range(8):
            i = 8 * k + j
            xi = j % XDEPTH
            oi = j % 2
            pp = (j // 4) % 2
            pc, b = split(i)

            @pl.when(i + XDEPTH - 1 < NITER)
            def _():
                issue_x(i + XDEPTH - 1, (j + XDEPTH - 1) % XDEPTH)

            if j % 4 == 0:
                wait_p(pp)

                @pl.when(pc + 1 < PCHUNKS)
                def _():
                    issue_p(pc + 1, 1 - pp)

            wait_x(xi)

            @pl.when(i >= 2)
            def _():
                wait_o(oi)

            xb = xbuf.at[xi]
            ob = obuf.at[oi]
            pb = pbuf.at[pp]

            @plsc.parallel_loop(0, CHUNK_EL, 16, unroll=UNROLL)
            def _(o, xb=xb, ob=ob, pb=pb):
                ob[pl.ds(o, 16)] = xb[pl.ds(o, 16)] + pb[pl.ds(o, 16)]

            issue_o(i, oi)
        return 0

    lax.fori_loop(0, NITER // 8, chunk_group, 0)

    wait_o(0)
    wait_o(1)


def kernel(x, pos_emb):
    dummy = jnp.zeros((CHUNK_EL,), jnp.float32)
    return _pos_add(x, pos_emb, dummy)


# speedup vs baseline: 1.2009x; 1.0026x over previous
import functools

import jax
import jax.numpy as jnp
from jax import lax
from jax.experimental import pallas as pl
from jax.experimental.pallas import tpu as pltpu
from jax.experimental.pallas import tpu_sc as plsc

BATCH, SEQ, DIM = 4, 8192, 768
NC, NS = 2, 16
NW = NC * NS
S_PER_W = SEQ // NW
CHUNK = 16
PCHUNKS = S_PER_W // CHUNK
NITER = PCHUNKS * BATCH
CHUNK_EL = CHUNK * DIM
UNROLL = 8
XDEPTH = 4

_mesh = plsc.VectorSubcoreMesh(core_axis_name="c", subcore_axis_name="s")


@functools.partial(
    pl.kernel,
    mesh=_mesh,
    out_type=jax.ShapeDtypeStruct((BATCH, SEQ, DIM), jnp.float32),
    scratch_types=[
        pltpu.VMEM((XDEPTH, CHUNK_EL), jnp.float32),
        pltpu.VMEM((2, CHUNK_EL), jnp.float32),
        pltpu.VMEM((4, CHUNK_EL), jnp.float32),
        pltpu.SemaphoreType.DMA,
        pltpu.SemaphoreType.DMA,
        pltpu.SemaphoreType.DMA,
        pltpu.SemaphoreType.DMA,
        pltpu.SemaphoreType.DMA,
        pltpu.SemaphoreType.DMA,
        pltpu.SemaphoreType.DMA,
        pltpu.SemaphoreType.DMA,
        pltpu.SemaphoreType.DMA,
        pltpu.SemaphoreType.DMA,
    ],
)
def _pos_add(x_hbm, pos_hbm, dummy_hbm, out_hbm, xbuf, pbuf, obuf,
             xsem0, xsem1, xsem2, xsem3, psem0, psem1,
             osem0, osem1, osem2, osem3):
    wid = lax.axis_index("s") * NC + lax.axis_index("c")
    s0 = wid * S_PER_W

    xsems = (xsem0, xsem1, xsem2, xsem3)
    psems = (psem0, psem1)
    osems = (osem0, osem1, osem2, osem3)

    def rows_loop(fn):
        def body(r, _):
            fn(r)
            return 0
        lax.fori_loop(0, CHUNK, body, 0)

    def split(i):
        return i // BATCH, lax.rem(i, BATCH)

    def issue_x(i, xi):
        pc, b = split(i)
        r0 = s0 + pc * CHUNK
        rows_loop(lambda r: pltpu.async_copy(
            x_hbm.at[b, r0 + r, :],
            xbuf.at[xi, pl.ds(r * DIM, DIM)], xsems[xi]))

    def wait_x(xi):
        pltpu.make_async_copy(dummy_hbm, xbuf.at[xi], xsems[xi]).wait()

    def issue_p(pc, pp):
        r0 = s0 + pc * CHUNK
        rows_loop(lambda r: pltpu.async_copy(
            pos_hbm.at[r0 + r, :], pbuf.at[pp, pl.ds(r * DIM, DIM)], psems[pp]))

    def wait_p(pp):
        pltpu.make_async_copy(dummy_hbm, pbuf.at[pp], psems[pp]).wait()

    def issue_o(i, oi):
        pc, b = split(i)
        r0 = s0 + pc * CHUNK
        rows_loop(lambda r: pltpu.async_copy(
            obuf.at[oi, pl.ds(r * DIM, DIM)],
            out_hbm.at[b, r0 + r, :], osems[oi]))

    def wait_o(oi):
        pltpu.make_async_copy(obuf.at[oi], dummy_hbm, osems[oi]).wait()

    issue_p(0, 0)
    for j in range(XDEPTH - 1):
        issue_x(j, j)

    def chunk_group(k, _):
        for j in range(8):
            i = 8 * k + j
            xi = j % XDEPTH
            oi = j % 4
            pp = (j // 4) % 2
            pc, b = split(i)

            @pl.when(i + XDEPTH - 1 < NITER)
            def _():
                issue_x(i + XDEPTH - 1, (j + XDEPTH - 1) % XDEPTH)

            if j % 4 == 0:
                wait_p(pp)

                @pl.when(pc + 1 < PCHUNKS)
                def _():
                    issue_p(pc + 1, 1 - pp)

            wait_x(xi)

            @pl.when(i >= 4)
            def _():
                wait_o(oi)

            xb = xbuf.at[xi]
            ob = obuf.at[oi]
            pb = pbuf.at[pp]

            @plsc.parallel_loop(0, CHUNK_EL, 16, unroll=UNROLL)
            def _(o, xb=xb, ob=ob, pb=pb):
                ob[pl.ds(o, 16)] = xb[pl.ds(o, 16)] + pb[pl.ds(o, 16)]

            issue_o(i, oi)
        return 0

    lax.fori_loop(0, NITER // 8, chunk_group, 0)

    for oi in range(4):
        wait_o(oi)


def kernel(x, pos_emb):
    dummy = jnp.zeros((CHUNK_EL,), jnp.float32)
    return _pos_add(x, pos_emb, dummy)
